# baseline (speedup 1.0000x reference)
import functools

import jax
import jax.numpy as jnp
from jax import lax
from jax.experimental import pallas as pl
from jax.experimental.pallas import tpu as pltpu
from jax.experimental.pallas import tpu_sc as plsc

N = 10000
NPAD = 10112
E = 320000
EL = 32768
H = 128
D_IN = 768

NC = 2
NS = 16
CH = 128
NCHUNK = 160
EPT = NCHUNK * CH
EPAD = NS * EPT - E
GRP = 40
NGRP = NCHUNK // GRP
ROWS_PT = NPAD // NS
CNTW = 128
ELPT = EL // NS
ELCH = 128
ELNCH = ELPT // ELCH

_mesh = plsc.VectorSubcoreMesh(core_axis_name="c", subcore_axis_name="s")


@functools.partial(
    pl.kernel,
    mesh=_mesh,
    out_type=jax.ShapeDtypeStruct((2 * NPAD, H), jnp.float32),
    scratch_types=[
        pltpu.VMEM((GRP, CH), jnp.int32),
        pltpu.VMEM((GRP, CH), jnp.int32),
        pltpu.VMEM((CH, H), jnp.float32),
        pltpu.VMEM((CH, H), jnp.float32),
        pltpu.VMEM_SHARED((NPAD, H), jnp.float32),
        pltpu.SemaphoreType.DMA,
        pltpu.SemaphoreType.DMA,
        pltpu.SemaphoreType.DMA,
        pltpu.SemaphoreType.DMA,
    ],
)
def _sc_seg_sum(x_hbm, e_hbm, zeros_hbm, out_hbm, src_v, dst_v,
                f0, f1, acc_sh, semg0, semg1, sems0, sems1):
    c = lax.axis_index("c")
    s = lax.axis_index("s")
    pltpu.sync_copy(zeros_hbm.at[pl.ds(s * ROWS_PT, ROWS_PT)],
                    acc_sh.at[pl.ds(s * ROWS_PT, ROWS_PT)])
    plsc.subcore_barrier()

    def outer(g, carry):
        pltpu.sync_copy(e_hbm.at[c, 0, s, pl.ds(g * GRP, GRP)], src_v)
        pltpu.sync_copy(e_hbm.at[c, 1, s, pl.ds(g * GRP, GRP)], dst_v)
        pltpu.async_copy(x_hbm.at[src_v.at[0]], f0, semg0)
        pltpu.async_copy(x_hbm.at[src_v.at[1]], f1, semg1)

        def body(t, carry2):
            j0 = 2 * t
            j1 = j0 + 1
            pltpu.make_async_copy(x_hbm.at[src_v.at[j0]], f0, semg0).wait()
            pltpu.async_copy(f0, acc_sh.at[dst_v.at[j0]], sems0, add=True)
            pltpu.make_async_copy(x_hbm.at[src_v.at[j1]], f1, semg1).wait()
            pltpu.make_async_copy(f0, acc_sh.at[dst_v.at[j0]], sems0).wait()

            @pl.when(j0 + 2 < GRP)
            def _():
                pltpu.async_copy(x_hbm.at[src_v.at[j0 + 2]], f0, semg0)

            pltpu.async_copy(f1, acc_sh.at[dst_v.at[j1]], sems1, add=True)
            pltpu.make_async_copy(f1, acc_sh.at[dst_v.at[j1]], sems1).wait()

            @pl.when(j1 + 2 < GRP)
            def _():
                pltpu.async_copy(x_hbm.at[src_v.at[j1 + 2]], f1, semg1)

            return carry2

        lax.fori_loop(0, GRP // 2, body, 0)
        return carry

    lax.fori_loop(0, NGRP, outer, 0)
    plsc.subcore_barrier()
    pltpu.sync_copy(acc_sh.at[pl.ds(s * ROWS_PT, ROWS_PT)],
                    out_hbm.at[pl.ds(c * NPAD + s * ROWS_PT, ROWS_PT)])


@functools.partial(
    pl.kernel,
    mesh=_mesh,
    out_type=jax.ShapeDtypeStruct((2 * NPAD, CNTW), jnp.float32),
    scratch_types=[
        pltpu.VMEM((GRP, CH), jnp.int32),
        pltpu.VMEM((CH, CNTW), jnp.float32),
        pltpu.VMEM_SHARED((NPAD, CNTW), jnp.float32),
        pltpu.SemaphoreType.DMA,
    ],
)
def _sc_counts(e_hbm, zeros_hbm, out_hbm, dst_v, ones_v, acc_sh, sem):
    c = lax.axis_index("c")
    s = lax.axis_index("s")
    pltpu.sync_copy(zeros_hbm.at[pl.ds(s * ROWS_PT, ROWS_PT)],
                    acc_sh.at[pl.ds(s * ROWS_PT, ROWS_PT)])
    one16 = jnp.ones((16,), jnp.float32)
    for r in range(CH):
        for k in range(CNTW // 16):
            ones_v[r, pl.ds(k * 16, 16)] = one16
    plsc.subcore_barrier()

    def outer(g, carry):
        pltpu.sync_copy(e_hbm.at[c, 1, s, pl.ds(g * GRP, GRP)], dst_v)

        def body(j, carry2):
            pltpu.sync_copy(ones_v, acc_sh.at[dst_v.at[j]], add=True)
            return carry2

        lax.fori_loop(0, GRP, body, 0)
        return carry

    lax.fori_loop(0, NGRP, outer, 0)
    plsc.subcore_barrier()
    pltpu.sync_copy(acc_sh.at[pl.ds(s * ROWS_PT, ROWS_PT)],
                    out_hbm.at[pl.ds(c * NPAD + s * ROWS_PT, ROWS_PT)])


@functools.partial(
    pl.kernel,
    mesh=_mesh,
    out_type=jax.ShapeDtypeStruct((2, EL, H), jnp.float32),
    scratch_types=[
        pltpu.VMEM((ELNCH, ELCH), jnp.int32),
        pltpu.VMEM((ELCH, H), jnp.float32),
        pltpu.VMEM((ELCH, H), jnp.float32),
        pltpu.SemaphoreType.DMA,
        pltpu.SemaphoreType.DMA,
    ],
)
def _sc_pair_gather(x_hbm, eli_hbm, out_hbm, idx_v, rows0, rows1, sem0, sem1):
    c = lax.axis_index("c")
    s = lax.axis_index("s")
    pltpu.sync_copy(eli_hbm.at[c, s], idx_v)
    pltpu.async_copy(x_hbm.at[idx_v.at[0]], rows0, sem0)
    pltpu.async_copy(x_hbm.at[idx_v.at[1]], rows1, sem1)

    def body(t, carry):
        j0 = 2 * t
        j1 = j0 + 1
        pltpu.make_async_copy(x_hbm.at[idx_v.at[j0]], rows0, sem0).wait()
        pltpu.sync_copy(rows0,
                        out_hbm.at[c, pl.ds(s * ELPT + j0 * ELCH, ELCH)])

        @pl.when(j0 + 2 < ELNCH)
        def _():
            pltpu.async_copy(x_hbm.at[idx_v.at[j0 + 2]], rows0, sem0)

        pltpu.make_async_copy(x_hbm.at[idx_v.at[j1]], rows1, sem1).wait()
        pltpu.sync_copy(rows1,
                        out_hbm.at[c, pl.ds(s * ELPT + j1 * ELCH, ELCH)])

        @pl.when(j1 + 2 < ELNCH)
        def _():
            pltpu.async_copy(x_hbm.at[idx_v.at[j1 + 2]], rows1, sem1)

        return carry

    lax.fori_loop(0, ELNCH // 2, body, 0)


_BLK = 1000
_BLKP = 632


def _init_sq_body(sqx_ref, w_ref, b_ref, memb_ref, o_ref):
    o_ref[...] = (
        lax.dot_general(sqx_ref[...], w_ref[...], (((1,), (1,)), ((), ())),
                        preferred_element_type=jnp.float32)
        + b_ref[...] + memb_ref[...]
    )


def _tc_init_sq(sq_x, lin_w, lin_b, movie_emb):
    return pl.pallas_call(
        _init_sq_body,
        grid=(N // _BLK,),
        in_specs=[
            pl.BlockSpec((_BLK, D_IN), lambda i: (i, 0)),
            pl.BlockSpec((H, D_IN), lambda i: (0, 0)),
            pl.BlockSpec((1, H), lambda i: (0, 0)),
            pl.BlockSpec((_BLK, H), lambda i: (i, 0)),
        ],
        out_specs=pl.BlockSpec((_BLK, H), lambda i: (i, 0)),
        out_shape=jax.ShapeDtypeStruct((N, H), jnp.float32),
    )(sq_x, lin_w, lin_b.reshape(1, H), movie_emb)


def _layer_body(acc_ref, cnt_ref, x_ref, wl_ref, bl_ref, wr_ref, o_ref,
                *, relu):
    inv = 1.0 / jnp.maximum(cnt_ref[..., 0:1], 1.0)
    mean = acc_ref[...] * inv
    y = (
        lax.dot_general(mean, wl_ref[0], (((1,), (1,)), ((), ())),
                        preferred_element_type=jnp.float32)
        + bl_ref[0]
        + lax.dot_general(x_ref[...], wr_ref[0], (((1,), (1,)), ((), ())),
                          preferred_element_type=jnp.float32)
    )
    if relu:
        y = jnp.maximum(y, 0.0)
    o_ref[...] = y


def _tc_layer(acc, cnt, x, wl, bl, wr, relu):
    nb = NPAD // _BLKP
    return pl.pallas_call(
        functools.partial(_layer_body, relu=relu),
        grid=(2, nb),
        in_specs=[
            pl.BlockSpec((_BLKP, H), lambda d, i: ((1 - d) * nb + i, 0)),
            pl.BlockSpec((_BLKP, CNTW), lambda d, i: ((1 - d) * nb + i, 0)),
            pl.BlockSpec((_BLKP, H), lambda d, i: (d * nb + i, 0)),
            pl.BlockSpec((1, H, H), lambda d, i: (d, 0, 0)),
            pl.BlockSpec((1, 1, H), lambda d, i: (d, 0, 0)),
            pl.BlockSpec((1, H, H), lambda d, i: (d, 0, 0)),
        ],
        out_specs=pl.BlockSpec((_BLKP, H), lambda d, i: (d * nb + i, 0)),
        out_shape=jax.ShapeDtypeStruct((2 * NPAD, H), jnp.float32),
    )(acc, cnt, x, wl, bl, wr)


_BLKC = 2048


def _dot_body(u_ref, m_ref, o_ref):
    o_ref[...] = jnp.sum(u_ref[0] * m_ref[0], axis=-1, keepdims=True)


def _tc_pair_dot(g):
    return pl.pallas_call(
        _dot_body,
        grid=(EL // _BLKC,),
        in_specs=[
            pl.BlockSpec((1, _BLKC, H), lambda i: (0, i, 0)),
            pl.BlockSpec((1, _BLKC, H), lambda i: (1, i, 0)),
        ],
        out_specs=pl.BlockSpec((_BLKC, 1), lambda i: (i, 0)),
        out_shape=jax.ShapeDtypeStruct((EL, 1), jnp.float32),
    )(g, g)


def kernel(params, sq_x, mq_node_id, sq_node_id, edge_index_mq_sq,
           edge_index_sq_mq, edge_label_index):
    p = params
    user_emb = p["user_emb"]
    movie_emb = p["movie_emb"]

    xsq0 = _tc_init_sq(sq_x, p["lin_W"], p["lin_b"], movie_emb)
    pad = jnp.zeros((NPAD - N, H), jnp.float32)
    x = jnp.concatenate([user_emb, pad, xsq0, pad], axis=0)

    pad_s = jnp.zeros((EPAD,), jnp.int32)
    pad_d = jnp.full((EPAD,), NPAD - 1, jnp.int32)
    src0 = jnp.concatenate([edge_index_mq_sq[0], pad_s])
    dst0 = jnp.concatenate([edge_index_mq_sq[1], pad_d])
    src1 = jnp.concatenate([edge_index_sq_mq[0], pad_s]) + NPAD
    dst1 = jnp.concatenate([edge_index_sq_mq[1], pad_d])
    e = jnp.stack([jnp.stack([src0, dst0]), jnp.stack([src1, dst1])])
    e = e.reshape(2, 2, NS, NCHUNK, CH)

    zeros_h = jnp.zeros((NPAD, H), jnp.float32)
    zeros_c = jnp.zeros((NPAD, CNTW), jnp.float32)
    cnt = _sc_counts(e, zeros_c)

    for i, layer in enumerate(p["convs"]):
        wl = jnp.stack([layer["sq_mq"]["W_l"], layer["mq_sq"]["W_l"]])
        bl = jnp.stack([layer["sq_mq"]["b_l"],
                        layer["mq_sq"]["b_l"]]).reshape(2, 1, H)
        wr = jnp.stack([layer["sq_mq"]["W_r"], layer["mq_sq"]["W_r"]])
        acc = _sc_seg_sum(x, e, zeros_h)
        x = _tc_layer(acc, cnt, x, wl, bl, wr, relu=(i == 0))

    eli_u = edge_label_index[0].reshape(NS, ELNCH, ELCH)
    eli_m = (edge_label_index[1] + NPAD).reshape(NS, ELNCH, ELCH)
    eli = jnp.stack([eli_u, eli_m])
    g = _sc_pair_gather(x, eli)
    out = _tc_pair_dot(g)
    return out.reshape(EL)


# speedup vs baseline: 2.3633x; 2.3633x over previous
import functools

import jax
import jax.numpy as jnp
from jax import lax
from jax.experimental import pallas as pl
from jax.experimental.pallas import tpu as pltpu
from jax.experimental.pallas import tpu_sc as plsc

N = 10000
NPAD = 10112
E = 320000
EL = 32768
H = 128
D_IN = 768

NC = 2
NS = 16
CH = 125
NCHUNK = 160
EPT = NCHUNK * CH
GRP = 40
NGRP = NCHUNK // GRP
ROWS_PT = NPAD // NS
CNTW = 128
ELPT = EL // NS
ELCH = 128
ELNCH = ELPT // ELCH

_mesh = plsc.VectorSubcoreMesh(core_axis_name="c", subcore_axis_name="s")


@functools.partial(
    pl.kernel,
    mesh=_mesh,
    out_type=jax.ShapeDtypeStruct((2 * NPAD, H), jnp.float32),
    scratch_types=[
        pltpu.VMEM((GRP, CH), jnp.int32),
        pltpu.VMEM((GRP, CH), jnp.int32),
        pltpu.VMEM((CH, H), jnp.float32),
        pltpu.VMEM((CH, H), jnp.float32),
        pltpu.VMEM_SHARED((NPAD, H), jnp.float32),
        pltpu.SemaphoreType.DMA,
        pltpu.SemaphoreType.DMA,
        pltpu.SemaphoreType.DMA,
        pltpu.SemaphoreType.DMA,
    ],
)
def _sc_seg_sum(x_hbm, e_hbm, zeros_hbm, out_hbm, src_v, dst_v,
                f0, f1, acc_sh, semg0, semg1, sems0, sems1):
    c = lax.axis_index("c")
    s = lax.axis_index("s")
    pltpu.sync_copy(zeros_hbm.at[pl.ds(s * ROWS_PT, ROWS_PT)],
                    acc_sh.at[pl.ds(s * ROWS_PT, ROWS_PT)])
    plsc.subcore_barrier()

    def outer(g, carry):
        pltpu.sync_copy(e_hbm.at[c, 0, s, pl.ds(g * GRP, GRP)], src_v)
        pltpu.sync_copy(e_hbm.at[c, 1, s, pl.ds(g * GRP, GRP)], dst_v)
        pltpu.async_copy(x_hbm.at[src_v.at[0]], f0, semg0)
        pltpu.async_copy(x_hbm.at[src_v.at[1]], f1, semg1)

        def body(t, carry2):
            j0 = 2 * t
            j1 = j0 + 1
            pltpu.make_async_copy(x_hbm.at[src_v.at[j0]], f0, semg0).wait()
            pltpu.async_copy(f0, acc_sh.at[dst_v.at[j0]], sems0, add=True)
            pltpu.make_async_copy(x_hbm.at[src_v.at[j1]], f1, semg1).wait()
            pltpu.make_async_copy(f0, acc_sh.at[dst_v.at[j0]], sems0).wait()

            @pl.when(j0 + 2 < GRP)
            def _():
                pltpu.async_copy(x_hbm.at[src_v.at[j0 + 2]], f0, semg0)

            pltpu.async_copy(f1, acc_sh.at[dst_v.at[j1]], sems1, add=True)
            pltpu.make_async_copy(f1, acc_sh.at[dst_v.at[j1]], sems1).wait()

            @pl.when(j1 + 2 < GRP)
            def _():
                pltpu.async_copy(x_hbm.at[src_v.at[j1 + 2]], f1, semg1)

            return carry2

        lax.fori_loop(0, GRP // 2, body, 0)
        return carry

    lax.fori_loop(0, NGRP, outer, 0)
    plsc.subcore_barrier()
    pltpu.sync_copy(acc_sh.at[pl.ds(s * ROWS_PT, ROWS_PT)],
                    out_hbm.at[pl.ds(c * NPAD + s * ROWS_PT, ROWS_PT)])


@functools.partial(
    pl.kernel,
    mesh=_mesh,
    out_type=jax.ShapeDtypeStruct((2 * NPAD, CNTW), jnp.float32),
    scratch_types=[
        pltpu.VMEM((GRP, CH), jnp.int32),
        pltpu.VMEM((CH, CNTW), jnp.float32),
        pltpu.VMEM_SHARED((NPAD, CNTW), jnp.float32),
        pltpu.SemaphoreType.DMA,
    ],
)
def _sc_counts(e_hbm, zeros_hbm, out_hbm, dst_v, ones_v, acc_sh, sem):
    c = lax.axis_index("c")
    s = lax.axis_index("s")
    pltpu.sync_copy(zeros_hbm.at[pl.ds(s * ROWS_PT, ROWS_PT)],
                    acc_sh.at[pl.ds(s * ROWS_PT, ROWS_PT)])
    one16 = jnp.ones((16,), jnp.float32)
    for r in range(CH):
        for k in range(CNTW // 16):
            ones_v[r, pl.ds(k * 16, 16)] = one16
    plsc.subcore_barrier()

    def outer(g, carry):
        pltpu.sync_copy(e_hbm.at[c, 1, s, pl.ds(g * GRP, GRP)], dst_v)

        def body(j, carry2):
            pltpu.sync_copy(ones_v, acc_sh.at[dst_v.at[j]], add=True)
            return carry2

        lax.fori_loop(0, GRP, body, 0)
        return carry

    lax.fori_loop(0, NGRP, outer, 0)
    plsc.subcore_barrier()
    pltpu.sync_copy(acc_sh.at[pl.ds(s * ROWS_PT, ROWS_PT)],
                    out_hbm.at[pl.ds(c * NPAD + s * ROWS_PT, ROWS_PT)])


@functools.partial(
    pl.kernel,
    mesh=_mesh,
    out_type=jax.ShapeDtypeStruct((2, EL, H), jnp.float32),
    scratch_types=[
        pltpu.VMEM((ELNCH, ELCH), jnp.int32),
        pltpu.VMEM((ELCH, H), jnp.float32),
        pltpu.VMEM((ELCH, H), jnp.float32),
        pltpu.SemaphoreType.DMA,
        pltpu.SemaphoreType.DMA,
    ],
)
def _sc_pair_gather(x_hbm, eli_hbm, out_hbm, idx_v, rows0, rows1, sem0, sem1):
    c = lax.axis_index("c")
    s = lax.axis_index("s")
    pltpu.sync_copy(eli_hbm.at[c, s], idx_v)
    pltpu.async_copy(x_hbm.at[idx_v.at[0]], rows0, sem0)
    pltpu.async_copy(x_hbm.at[idx_v.at[1]], rows1, sem1)

    def body(t, carry):
        j0 = 2 * t
        j1 = j0 + 1
        pltpu.make_async_copy(x_hbm.at[idx_v.at[j0]], rows0, sem0).wait()
        pltpu.sync_copy(rows0,
                        out_hbm.at[c, pl.ds(s * ELPT + j0 * ELCH, ELCH)])

        @pl.when(j0 + 2 < ELNCH)
        def _():
            pltpu.async_copy(x_hbm.at[idx_v.at[j0 + 2]], rows0, sem0)

        pltpu.make_async_copy(x_hbm.at[idx_v.at[j1]], rows1, sem1).wait()
        pltpu.sync_copy(rows1,
                        out_hbm.at[c, pl.ds(s * ELPT + j1 * ELCH, ELCH)])

        @pl.when(j1 + 2 < ELNCH)
        def _():
            pltpu.async_copy(x_hbm.at[idx_v.at[j1 + 2]], rows1, sem1)

        return carry

    lax.fori_loop(0, ELNCH // 2, body, 0)


_BLK = 1000
_BLKP = 632


def _init_sq_body(sqx_ref, w_ref, b_ref, memb_ref, o_ref):
    o_ref[...] = (
        lax.dot_general(sqx_ref[...], w_ref[...], (((1,), (1,)), ((), ())),
                        preferred_element_type=jnp.float32)
        + b_ref[...] + memb_ref[...]
    )


def _tc_init_sq(sq_x, lin_w, lin_b, movie_emb):
    return pl.pallas_call(
        _init_sq_body,
        grid=(N // _BLK,),
        in_specs=[
            pl.BlockSpec((_BLK, D_IN), lambda i: (i, 0)),
            pl.BlockSpec((H, D_IN), lambda i: (0, 0)),
            pl.BlockSpec((1, H), lambda i: (0, 0)),
            pl.BlockSpec((_BLK, H), lambda i: (i, 0)),
        ],
        out_specs=pl.BlockSpec((_BLK, H), lambda i: (i, 0)),
        out_shape=jax.ShapeDtypeStruct((N, H), jnp.float32),
    )(sq_x, lin_w, lin_b.reshape(1, H), movie_emb)


def _layer_body(acc_ref, cnt_ref, x_ref, wl_ref, bl_ref, wr_ref, o_ref,
                *, relu):
    inv = 1.0 / jnp.maximum(cnt_ref[..., 0:1], 1.0)
    mean = acc_ref[...] * inv
    y = (
        lax.dot_general(mean, wl_ref[0], (((1,), (1,)), ((), ())),
                        preferred_element_type=jnp.float32)
        + bl_ref[0]
        + lax.dot_general(x_ref[...], wr_ref[0], (((1,), (1,)), ((), ())),
                          preferred_element_type=jnp.float32)
    )
    if relu:
        y = jnp.maximum(y, 0.0)
    o_ref[...] = y


def _tc_layer(acc, cnt, x, wl, bl, wr, relu):
    nb = NPAD // _BLKP
    return pl.pallas_call(
        functools.partial(_layer_body, relu=relu),
        grid=(2, nb),
        in_specs=[
            pl.BlockSpec((_BLKP, H), lambda d, i: ((1 - d) * nb + i, 0)),
            pl.BlockSpec((_BLKP, CNTW), lambda d, i: ((1 - d) * nb + i, 0)),
            pl.BlockSpec((_BLKP, H), lambda d, i: (d * nb + i, 0)),
            pl.BlockSpec((1, H, H), lambda d, i: (d, 0, 0)),
            pl.BlockSpec((1, 1, H), lambda d, i: (d, 0, 0)),
            pl.BlockSpec((1, H, H), lambda d, i: (d, 0, 0)),
        ],
        out_specs=pl.BlockSpec((_BLKP, H), lambda d, i: (d * nb + i, 0)),
        out_shape=jax.ShapeDtypeStruct((2 * NPAD, H), jnp.float32),
    )(acc, cnt, x, wl, bl, wr)


_BLKC = 2048


def _dot_body(u_ref, m_ref, o_ref):
    o_ref[...] = jnp.sum(u_ref[0] * m_ref[0], axis=-1, keepdims=True)


def _tc_pair_dot(g):
    return pl.pallas_call(
        _dot_body,
        grid=(EL // _BLKC,),
        in_specs=[
            pl.BlockSpec((1, _BLKC, H), lambda i: (0, i, 0)),
            pl.BlockSpec((1, _BLKC, H), lambda i: (1, i, 0)),
        ],
        out_specs=pl.BlockSpec((_BLKC, 1), lambda i: (i, 0)),
        out_shape=jax.ShapeDtypeStruct((EL, 1), jnp.float32),
    )(g, g)


def kernel(params, sq_x, mq_node_id, sq_node_id, edge_index_mq_sq,
           edge_index_sq_mq, edge_label_index):
    p = params
    user_emb = p["user_emb"]
    movie_emb = p["movie_emb"]

    xsq0 = _tc_init_sq(sq_x, p["lin_W"], p["lin_b"], movie_emb)
    pad = jnp.zeros((NPAD - N, H), jnp.float32)
    x = jnp.concatenate([user_emb, pad, xsq0, pad], axis=0)

    src0 = edge_index_mq_sq[0]
    dst0 = edge_index_mq_sq[1]
    src1 = edge_index_sq_mq[0] + NPAD
    dst1 = edge_index_sq_mq[1]
    e = jnp.stack([jnp.stack([src0, dst0]), jnp.stack([src1, dst1])])
    e = e.reshape(2, 2, NS, NCHUNK, CH)

    zeros_h = jnp.zeros((NPAD, H), jnp.float32)
    zeros_c = jnp.zeros((NPAD, CNTW), jnp.float32)
    cnt = _sc_counts(e, zeros_c)

    for i, layer in enumerate(p["convs"]):
        wl = jnp.stack([layer["sq_mq"]["W_l"], layer["mq_sq"]["W_l"]])
        bl = jnp.stack([layer["sq_mq"]["b_l"],
                        layer["mq_sq"]["b_l"]]).reshape(2, 1, H)
        wr = jnp.stack([layer["sq_mq"]["W_r"], layer["mq_sq"]["W_r"]])
        acc = _sc_seg_sum(x, e, zeros_h)
        x = _tc_layer(acc, cnt, x, wl, bl, wr, relu=(i == 0))

    eli_u = edge_label_index[0].reshape(NS, ELNCH, ELCH)
    eli_m = (edge_label_index[1] + NPAD).reshape(NS, ELNCH, ELCH)
    eli = jnp.stack([eli_u, eli_m])
    g = _sc_pair_gather(x, eli)
    out = _tc_pair_dot(g)
    return out.reshape(EL)
